# baseline (speedup 1.0000x reference)
import jax
import jax.numpy as jnp
from jax.experimental import pallas as pl

_POS_RFACTOR = 8
_RPAD = 32
_REP = 64
_NJ = 2


def _embed_add_kernel(idx_ref, tabT_ref, x_ref, o_ref):
    g = idx_ref.shape[-1]
    idx_row = idx_ref[0] // _POS_RFACTOR
    iota_r = jax.lax.broadcasted_iota(jnp.int32, (_RPAD, g), 0)
    oh = (iota_r == idx_row).astype(jnp.float32)
    e = jnp.dot(tabT_ref[...], oh, preferred_element_type=jnp.float32)
    o_ref[0] = x_ref[0] + e[:, :, None]


def kernel(x, pos, table):
    b, d, h = x.shape
    hp = pos.shape[2]
    rows = table.shape[0]
    g = hp // _NJ
    idx = pos[0, :, :, 0].astype(jnp.int32).reshape(b * _NJ, 1, g)
    tab_t = jnp.zeros((d, _RPAD), jnp.float32).at[:, :rows].set(table.T)
    x4 = x.reshape(b, d, hp, _REP)
    out = pl.pallas_call(
        _embed_add_kernel,
        grid=(b, _NJ),
        in_specs=[
            pl.BlockSpec((1, 1, g), lambda bi, ji: (bi * _NJ + ji, 0, 0)),
            pl.BlockSpec((d, _RPAD), lambda bi, ji: (0, 0)),
            pl.BlockSpec((1, d, g, _REP), lambda bi, ji: (bi, 0, ji, 0)),
        ],
        out_specs=pl.BlockSpec((1, d, g, _REP), lambda bi, ji: (bi, 0, ji, 0)),
        out_shape=jax.ShapeDtypeStruct(x4.shape, x.dtype),
    )(idx, tab_t, x4)
    return out.reshape(b, d, h)


# speedup vs baseline: 5.4956x; 5.4956x over previous
import jax
import jax.numpy as jnp
from jax.experimental import pallas as pl

_POS_RFACTOR = 8
_RPAD = 32
_REP = 64
_HBLK = 2048


def _embed_add_kernel(idxh_ref, tabT_ref, x_ref, o_ref):
    idx_h = idxh_ref[0]
    iota_r = jax.lax.broadcasted_iota(jnp.int32, (_RPAD, _HBLK), 0)
    m = (iota_r == idx_h).astype(jnp.float32)
    emb = jnp.dot(tabT_ref[...], m, preferred_element_type=jnp.float32)
    o_ref[0] = x_ref[0] + emb


def kernel(x, pos, table):
    b, d, h = x.shape
    hp = pos.shape[2]
    rows = table.shape[0]
    idx = pos[0, :, :, 0].astype(jnp.int32) // _POS_RFACTOR
    idx_h = jnp.broadcast_to(idx[:, :, None], (b, hp, h // hp)).reshape(b, 1, h)
    tab_t = jnp.zeros((d, _RPAD), jnp.float32).at[:, :rows].set(table.T)
    return pl.pallas_call(
        _embed_add_kernel,
        grid=(b, h // _HBLK),
        in_specs=[
            pl.BlockSpec((1, 1, _HBLK), lambda bi, ji: (bi, 0, ji)),
            pl.BlockSpec((d, _RPAD), lambda bi, ji: (0, 0)),
            pl.BlockSpec((1, d, _HBLK), lambda bi, ji: (bi, 0, ji)),
        ],
        out_specs=pl.BlockSpec((1, d, _HBLK), lambda bi, ji: (bi, 0, ji)),
        out_shape=jax.ShapeDtypeStruct(x.shape, x.dtype),
    )(idx_h, tab_t, x)


# speedup vs baseline: 7.9130x; 1.4399x over previous
import jax
import jax.numpy as jnp
from jax.experimental import pallas as pl

_POS_RFACTOR = 8
_RPAD = 32
_REP = 64
_HBLK = 4096


def _embed_add_kernel(idxh_ref, tabT_ref, x_ref, o_ref):
    idx_h = idxh_ref[0]
    iota_r = jax.lax.broadcasted_iota(jnp.int32, (_RPAD, _HBLK), 0)
    m = (iota_r == idx_h).astype(jnp.float32)
    emb = jnp.dot(tabT_ref[...], m, preferred_element_type=jnp.float32)
    o_ref[0] = x_ref[0] + emb


def kernel(x, pos, table):
    b, d, h = x.shape
    hp = pos.shape[2]
    rows = table.shape[0]
    idx = pos[0, :, :, 0].astype(jnp.int32) // _POS_RFACTOR
    idx_h = jnp.broadcast_to(idx[:, :, None], (b, hp, h // hp)).reshape(b, 1, h)
    tab_t = jnp.zeros((d, _RPAD), jnp.float32).at[:, :rows].set(table.T)
    return pl.pallas_call(
        _embed_add_kernel,
        grid=(b, h // _HBLK),
        in_specs=[
            pl.BlockSpec((1, 1, _HBLK), lambda bi, ji: (bi, 0, ji)),
            pl.BlockSpec((d, _RPAD), lambda bi, ji: (0, 0)),
            pl.BlockSpec((1, d, _HBLK), lambda bi, ji: (bi, 0, ji)),
        ],
        out_specs=pl.BlockSpec((1, d, _HBLK), lambda bi, ji: (bi, 0, ji)),
        out_shape=jax.ShapeDtypeStruct(x.shape, x.dtype),
    )(idx_h, tab_t, x)


# speedup vs baseline: 8.9241x; 1.1278x over previous
import jax
import jax.numpy as jnp
from jax.experimental import pallas as pl

_POS_RFACTOR = 8
_RPAD = 32
_REP = 64
_HBLK = 8192


def _embed_add_kernel(idxh_ref, tabT_ref, x_ref, o_ref):
    idx_h = idxh_ref[0]
    iota_r = jax.lax.broadcasted_iota(jnp.int32, (_RPAD, _HBLK), 0)
    m = (iota_r == idx_h).astype(jnp.float32)
    emb = jnp.dot(tabT_ref[...], m, preferred_element_type=jnp.float32)
    o_ref[0] = x_ref[0] + emb


def kernel(x, pos, table):
    b, d, h = x.shape
    hp = pos.shape[2]
    rows = table.shape[0]
    idx = pos[0, :, :, 0].astype(jnp.int32) // _POS_RFACTOR
    idx_h = jnp.broadcast_to(idx[:, :, None], (b, hp, h // hp)).reshape(b, 1, h)
    tab_t = jnp.zeros((d, _RPAD), jnp.float32).at[:, :rows].set(table.T)
    return pl.pallas_call(
        _embed_add_kernel,
        grid=(b, h // _HBLK),
        in_specs=[
            pl.BlockSpec((1, 1, _HBLK), lambda bi, ji: (bi, 0, ji)),
            pl.BlockSpec((d, _RPAD), lambda bi, ji: (0, 0)),
            pl.BlockSpec((1, d, _HBLK), lambda bi, ji: (bi, 0, ji)),
        ],
        out_specs=pl.BlockSpec((1, d, _HBLK), lambda bi, ji: (bi, 0, ji)),
        out_shape=jax.ShapeDtypeStruct(x.shape, x.dtype),
    )(idx_h, tab_t, x)


# speedup vs baseline: 9.0055x; 1.0091x over previous
import jax
import jax.numpy as jnp
from jax.experimental import pallas as pl

_POS_RFACTOR = 8
_RPAD = 32
_NB = 2


def _embed_add_kernel(idxh_ref, tabT_ref, x_ref, o_ref):
    hn = x_ref.shape[-1]
    iota_r = jax.lax.broadcasted_iota(jnp.int32, (_RPAD, hn), 0)
    for k in range(_NB):
        m = (iota_r == idxh_ref[k]).astype(jnp.float32)
        emb = jnp.dot(tabT_ref[...], m, preferred_element_type=jnp.float32)
        o_ref[k] = x_ref[k] + emb


def kernel(x, pos, table):
    b, d, h = x.shape
    hp = pos.shape[2]
    rows = table.shape[0]
    idx = pos[0, :, :, 0].astype(jnp.int32) // _POS_RFACTOR
    idx_h = jnp.broadcast_to(idx[:, :, None], (b, hp, h // hp)).reshape(b, 1, h)
    tab_t = jnp.zeros((d, _RPAD), jnp.float32).at[:, :rows].set(table.T)
    return pl.pallas_call(
        _embed_add_kernel,
        grid=(b // _NB,),
        in_specs=[
            pl.BlockSpec((_NB, 1, h), lambda bi: (bi, 0, 0)),
            pl.BlockSpec((d, _RPAD), lambda bi: (0, 0)),
            pl.BlockSpec((_NB, d, h), lambda bi: (bi, 0, 0)),
        ],
        out_specs=pl.BlockSpec((_NB, d, h), lambda bi: (bi, 0, 0)),
        out_shape=jax.ShapeDtypeStruct(x.shape, x.dtype),
    )(idx_h, tab_t, x)
